# baseline (speedup 1.0000x reference)
import jax
import jax.numpy as jnp
from jax import lax
from jax.experimental import pallas as pl
from jax.experimental.pallas import tpu as pltpu


def _route_kernel(eit_ref, ewt_ref, call_ref, used_ref, nu_ref):
    ne, t = call_ref.shape
    k = eit_ref.shape[0]
    io_e = lax.broadcasted_iota(jnp.int32, (ne, t), 0)
    call = jnp.zeros((ne, t), jnp.float32)
    routed = jnp.zeros((ne, t), jnp.float32)
    for kk in range(k):
        oh = (io_e == jnp.broadcast_to(eit_ref[kk:kk + 1, :], (ne, t)))
        ohf = oh.astype(jnp.float32)
        call += ohf * jnp.broadcast_to(ewt_ref[kk:kk + 1, :], (ne, t))
        routed += ohf
    call_ref[...] = call
    m_col = (jnp.sum(routed, axis=1, keepdims=True) > 0.0)
    m_f = m_col.astype(jnp.float32)
    tri = (lax.broadcasted_iota(jnp.int32, (ne, ne), 0)
           >= lax.broadcasted_iota(jnp.int32, (ne, ne), 1)
           ).astype(jnp.float32)
    pos_f = jnp.dot(tri, m_f, preferred_element_type=jnp.float32) - 1.0
    pos = pos_f.astype(jnp.int32)
    io_slot = lax.broadcasted_iota(jnp.int32, (ne, ne), 1)
    sel = jnp.logical_and(jnp.broadcast_to(pos, (ne, ne)) == io_slot,
                          jnp.broadcast_to(m_col, (ne, ne)))
    eids = lax.broadcasted_iota(jnp.int32, (ne, ne), 0).astype(jnp.float32)
    used_f = jnp.sum(sel.astype(jnp.float32) * eids, axis=0, keepdims=True)
    used_ref[...] = used_f.astype(jnp.int32)
    nu_ref[...] = jnp.sum(m_f, axis=0, keepdims=True).astype(jnp.int32)


def _ffn_kernel(used_ref, nu_ref, x_ref, call_ref, w1_ref, w2_ref, w3_ref,
                out_ref):
    i = pl.program_id(0)
    ic = pl.program_id(1)

    @pl.when(jnp.logical_and(i == 0, ic == 0))
    def _init():
        out_ref[...] = jnp.zeros_like(out_ref)

    @pl.when(i < nu_ref[0, 0])
    def _body():
        x = x_ref[...]
        g = jnp.dot(x, w1_ref[0], preferred_element_type=jnp.float32)
        u = jnp.dot(x, w3_ref[0], preferred_element_type=jnp.float32)
        h = g * jax.lax.logistic(g) * u
        y = jnp.dot(h, w2_ref[0], preferred_element_type=jnp.float32)
        c = call_ref[used_ref[0, i], :]
        out_ref[...] += c[:, None] * y


def _expert_block(i, used, nu):
    return used[0, jnp.minimum(i, nu[0, 0] - 1)]


def _chunk_block(i, ic, nu):
    return jnp.where(i < nu[0, 0], ic, 1)


@jax.jit
def kernel(x, expert_indices, expert_weights, w1_stacked, w2_stacked,
           w3_stacked):
    t, h = x.shape
    e, _, inter = w1_stacked.shape
    k = expert_indices.shape[1]
    n = t * k

    eit = expert_indices.astype(jnp.int32).T
    ewt = expert_weights.T
    call, used, nu = pl.pallas_call(
        _route_kernel,
        out_shape=[
            jax.ShapeDtypeStruct((e, t), jnp.float32),
            jax.ShapeDtypeStruct((1, e), jnp.int32),
            jax.ShapeDtypeStruct((1, 1), jnp.int32),
        ],
    )(eit, ewt)

    grid_spec = pltpu.PrefetchScalarGridSpec(
        num_scalar_prefetch=2,
        grid=(n, 2),
        in_specs=[
            pl.BlockSpec((t, h), lambda i, ic, used, nu: (0, 0)),
            pl.BlockSpec((e, t), lambda i, ic, used, nu: (0, 0)),
            pl.BlockSpec((1, h, inter // 2),
                         lambda i, ic, used, nu: (_expert_block(i, used, nu), 0,
                                                  _chunk_block(i, ic, nu))),
            pl.BlockSpec((1, inter // 2, h),
                         lambda i, ic, used, nu: (_expert_block(i, used, nu),
                                                  _chunk_block(i, ic, nu), 0)),
            pl.BlockSpec((1, h, inter // 2),
                         lambda i, ic, used, nu: (_expert_block(i, used, nu), 0,
                                                  _chunk_block(i, ic, nu))),
        ],
        out_specs=pl.BlockSpec((t, h), lambda i, ic, used, nu: (0, 0)),
    )
    return pl.pallas_call(
        _ffn_kernel,
        grid_spec=grid_spec,
        out_shape=jax.ShapeDtypeStruct((t, h), jnp.float32),
    )(used, nu, x, call, w1_stacked, w2_stacked, w3_stacked)


# speedup vs baseline: 1.0350x; 1.0350x over previous
import functools

import jax
import jax.numpy as jnp
from jax import lax
from jax.experimental import pallas as pl
from jax.experimental.pallas import tpu as pltpu


def _moe_kernel(eit_vm, ewt_vm, eit_sm, x_ref, w1_hbm, w2_hbm, w3_hbm,
                out_ref, call_v, marks, used_s, cnt, w1b, w2b, w3b, sems):
    ne, t = call_v.shape
    k = eit_vm.shape[0]

    io_e = lax.broadcasted_iota(jnp.int32, (ne, t), 0)
    call = jnp.zeros((ne, t), jnp.float32)
    for kk in range(k):
        oh = (io_e == jnp.broadcast_to(eit_vm[kk:kk + 1, :], (ne, t)))
        call += oh.astype(jnp.float32) * jnp.broadcast_to(
            ewt_vm[kk:kk + 1, :], (ne, t))
    call_v[...] = call

    def _zero(i, c):
        marks[0, i] = 0
        return c

    lax.fori_loop(0, ne, _zero, 0)

    def _mark(j, c):
        for kk in range(k):
            marks[0, eit_sm[kk, j]] = 1
        return c

    lax.fori_loop(0, t, _mark, 0)
    cnt[0] = 0

    def _compact(ee, c):
        m = marks[0, ee]
        idx = cnt[0]

        @pl.when(m == 1)
        def _w():
            used_s[0, idx] = ee

        cnt[0] = idx + m
        return c

    lax.fori_loop(0, ne, _compact, 0)
    nu = cnt[0]

    def _copies(e, slot):
        return (
            pltpu.make_async_copy(w1_hbm.at[e], w1b.at[slot], sems.at[3 * slot]),
            pltpu.make_async_copy(w2_hbm.at[e], w2b.at[slot], sems.at[3 * slot + 1]),
            pltpu.make_async_copy(w3_hbm.at[e], w3b.at[slot], sems.at[3 * slot + 2]),
        )

    def _fetch(e, slot):
        for cp in _copies(e, slot):
            cp.start()

    def _compute(j, slot):
        e = used_s[0, j]
        for cp in _copies(e, slot):
            cp.wait()
        x = x_ref[...]
        g = jnp.dot(x, w1b[slot], preferred_element_type=jnp.float32)
        u = jnp.dot(x, w3b[slot], preferred_element_type=jnp.float32)
        h = g * jax.lax.logistic(g) * u
        y = jnp.dot(h, w2b[slot], preferred_element_type=jnp.float32)
        c = call_v[e, :]
        out_ref[...] += c[:, None] * y

        @pl.when(j + 2 < nu)
        def _next():
            _fetch(used_s[0, j + 2], slot)

    out_ref[...] = jnp.zeros_like(out_ref)
    _fetch(used_s[0, 0], 0)

    @pl.when(nu > 1)
    def _pro1():
        _fetch(used_s[0, 1], 1)

    def _pair(jj, c):
        j = jj * 2
        _compute(j, 0)

        @pl.when(j + 1 < nu)
        def _odd():
            _compute(j + 1, 1)

        return c

    lax.fori_loop(0, (nu + 1) // 2, _pair, 0)


@jax.jit
def kernel(x, expert_indices, expert_weights, w1_stacked, w2_stacked,
           w3_stacked):
    t, h = x.shape
    e, _, inter = w1_stacked.shape
    k = expert_indices.shape[1]

    eit = expert_indices.astype(jnp.int32).T
    ewt = expert_weights.T

    return pl.pallas_call(
        _moe_kernel,
        in_specs=[
            pl.BlockSpec(memory_space=pltpu.MemorySpace.VMEM),
            pl.BlockSpec(memory_space=pltpu.MemorySpace.VMEM),
            pl.BlockSpec(memory_space=pltpu.MemorySpace.SMEM),
            pl.BlockSpec(memory_space=pltpu.MemorySpace.VMEM),
            pl.BlockSpec(memory_space=pltpu.MemorySpace.HBM),
            pl.BlockSpec(memory_space=pltpu.MemorySpace.HBM),
            pl.BlockSpec(memory_space=pltpu.MemorySpace.HBM),
        ],
        out_specs=pl.BlockSpec(memory_space=pltpu.MemorySpace.VMEM),
        out_shape=jax.ShapeDtypeStruct((t, h), jnp.float32),
        scratch_shapes=[
            pltpu.VMEM((e, t), jnp.float32),
            pltpu.SMEM((1, e), jnp.int32),
            pltpu.SMEM((1, e), jnp.int32),
            pltpu.SMEM((1,), jnp.int32),
            pltpu.VMEM((2, h, inter), jnp.float32),
            pltpu.VMEM((2, inter, h), jnp.float32),
            pltpu.VMEM((2, h, inter), jnp.float32),
            pltpu.SemaphoreType.DMA((6,)),
        ],
    )(eit, ewt, eit, x, w1_stacked, w2_stacked, w3_stacked)


# speedup vs baseline: 1.0471x; 1.0116x over previous
import functools

import jax
import jax.numpy as jnp
from jax import lax
from jax.experimental import pallas as pl
from jax.experimental.pallas import tpu as pltpu


def _moe_kernel(eit_vm, ewt_vm, eit_sm, x_ref, w1_hbm, w2_hbm, w3_hbm,
                out_ref, call_v, marks, used_s, cnt, w1b, w2b, w3b, sems):
    ne, t = call_v.shape
    k = eit_vm.shape[0]

    e0 = eit_sm[0, 0]

    def _copies(e, slot):
        return (
            pltpu.make_async_copy(w1_hbm.at[e], w1b.at[slot], sems.at[3 * slot]),
            pltpu.make_async_copy(w2_hbm.at[e], w2b.at[slot], sems.at[3 * slot + 1]),
            pltpu.make_async_copy(w3_hbm.at[e], w3b.at[slot], sems.at[3 * slot + 2]),
        )

    def _fetch(e, slot):
        for cp in _copies(e, slot):
            cp.start()

    _fetch(e0, 0)

    io_e = lax.broadcasted_iota(jnp.int32, (ne, t), 0)
    call = jnp.zeros((ne, t), jnp.float32)
    for kk in range(k):
        oh = (io_e == jnp.broadcast_to(eit_vm[kk:kk + 1, :], (ne, t)))
        call += oh.astype(jnp.float32) * jnp.broadcast_to(
            ewt_vm[kk:kk + 1, :], (ne, t))
    call_v[...] = call

    def _zero(i, c):
        marks[0, i] = 0
        return c

    lax.fori_loop(0, ne, _zero, 0)

    def _mark(j, c):
        for kk in range(k):
            marks[0, eit_sm[kk, j]] = 1
        return c

    lax.fori_loop(0, t, _mark, 0)
    marks[0, e0] = 0
    used_s[0, 0] = e0
    cnt[0] = 1

    def _compact(ee, c):
        m = marks[0, ee]
        idx = cnt[0]

        @pl.when(m == 1)
        def _w():
            used_s[0, idx] = ee

        cnt[0] = idx + m
        return c

    lax.fori_loop(0, ne, _compact, 0)
    nu = cnt[0]

    def _compute(j, slot):
        e = used_s[0, j]
        for cp in _copies(e, slot):
            cp.wait()
        x = x_ref[...]
        g = jnp.dot(x, w1b[slot], preferred_element_type=jnp.float32)
        u = jnp.dot(x, w3b[slot], preferred_element_type=jnp.float32)
        h = g * jax.lax.logistic(g) * u
        y = jnp.dot(h, w2b[slot], preferred_element_type=jnp.float32)
        c = call_v[e, :]
        out_ref[...] += c[:, None] * y

        @pl.when(j + 2 < nu)
        def _next():
            _fetch(used_s[0, j + 2], slot)

    out_ref[...] = jnp.zeros_like(out_ref)

    @pl.when(nu > 1)
    def _pro1():
        _fetch(used_s[0, 1], 1)

    def _pair(jj, c):
        j = jj * 2
        _compute(j, 0)

        @pl.when(j + 1 < nu)
        def _odd():
            _compute(j + 1, 1)

        return c

    lax.fori_loop(0, (nu + 1) // 2, _pair, 0)


@jax.jit
def kernel(x, expert_indices, expert_weights, w1_stacked, w2_stacked,
           w3_stacked):
    t, h = x.shape
    e, _, inter = w1_stacked.shape
    k = expert_indices.shape[1]

    eit = expert_indices.astype(jnp.int32).T
    ewt = expert_weights.T

    return pl.pallas_call(
        _moe_kernel,
        in_specs=[
            pl.BlockSpec(memory_space=pltpu.MemorySpace.VMEM),
            pl.BlockSpec(memory_space=pltpu.MemorySpace.VMEM),
            pl.BlockSpec(memory_space=pltpu.MemorySpace.SMEM),
            pl.BlockSpec(memory_space=pltpu.MemorySpace.VMEM),
            pl.BlockSpec(memory_space=pltpu.MemorySpace.HBM),
            pl.BlockSpec(memory_space=pltpu.MemorySpace.HBM),
            pl.BlockSpec(memory_space=pltpu.MemorySpace.HBM),
        ],
        out_specs=pl.BlockSpec(memory_space=pltpu.MemorySpace.VMEM),
        out_shape=jax.ShapeDtypeStruct((t, h), jnp.float32),
        scratch_shapes=[
            pltpu.VMEM((e, t), jnp.float32),
            pltpu.SMEM((1, e), jnp.int32),
            pltpu.SMEM((1, e), jnp.int32),
            pltpu.SMEM((1,), jnp.int32),
            pltpu.VMEM((2, h, inter), jnp.float32),
            pltpu.VMEM((2, inter, h), jnp.float32),
            pltpu.VMEM((2, h, inter), jnp.float32),
            pltpu.SemaphoreType.DMA((6,)),
        ],
    )(eit, ewt, eit, x, w1_stacked, w2_stacked, w3_stacked)


# speedup vs baseline: 1.0480x; 1.0009x over previous
import functools

import jax
import jax.numpy as jnp
from jax import lax
from jax.experimental import pallas as pl
from jax.experimental.pallas import tpu as pltpu


def _moe_kernel(eit_vm, ewt_vm, eit_sm, x_ref, w1_hbm, w2_hbm, w3_hbm,
                out_ref, call_v, marks, used_s, cnt, w1b, w2b, w3b, sems):
    ne, t = call_v.shape
    k = eit_vm.shape[0]

    e0 = eit_sm[0, 0]

    def _copies(e, slot):
        return (
            pltpu.make_async_copy(w1_hbm.at[e], w1b.at[slot], sems.at[3 * slot]),
            pltpu.make_async_copy(w2_hbm.at[e], w2b.at[slot], sems.at[3 * slot + 1]),
            pltpu.make_async_copy(w3_hbm.at[e], w3b.at[slot], sems.at[3 * slot + 2]),
        )

    def _fetch(e, slot):
        for cp in _copies(e, slot):
            cp.start()

    _fetch(e0, 0)

    io_e = lax.broadcasted_iota(jnp.int32, (ne, t), 0)
    call = jnp.zeros((ne, t), jnp.float32)
    for kk in range(k):
        oh = (io_e == jnp.broadcast_to(eit_vm[kk:kk + 1, :], (ne, t)))
        call += oh.astype(jnp.float32) * jnp.broadcast_to(
            ewt_vm[kk:kk + 1, :], (ne, t))
    call_v[...] = call

    def _zero(i, c):
        marks[0, i] = 0
        return c

    lax.fori_loop(0, ne, _zero, 0)

    def _mark(j, c):
        for kk in range(k):
            marks[0, eit_sm[kk, j]] = 1
        return c

    lax.fori_loop(0, t, _mark, 0)
    marks[0, e0] = 0
    used_s[0, 0] = e0
    cnt[0] = 1

    def _compact(ee, c):
        m = marks[0, ee]
        idx = cnt[0]

        @pl.when(m == 1)
        def _w():
            used_s[0, idx] = ee

            @pl.when(idx == 1)
            def _f1():
                _fetch(ee, 1)

        cnt[0] = idx + m
        return c

    lax.fori_loop(0, ne, _compact, 0)
    nu = cnt[0]

    def _compute(j, slot):
        e = used_s[0, j]
        for cp in _copies(e, slot):
            cp.wait()
        x = x_ref[...]
        g = jnp.dot(x, w1b[slot], preferred_element_type=jnp.float32)
        u = jnp.dot(x, w3b[slot], preferred_element_type=jnp.float32)
        h = g * jax.lax.logistic(g) * u
        y = jnp.dot(h, w2b[slot], preferred_element_type=jnp.float32)
        c = call_v[e, :]
        out_ref[...] += c[:, None] * y

        @pl.when(j + 2 < nu)
        def _next():
            _fetch(used_s[0, j + 2], slot)

    out_ref[...] = jnp.zeros_like(out_ref)

    def _pair(jj, c):
        j = jj * 2
        _compute(j, 0)

        @pl.when(j + 1 < nu)
        def _odd():
            _compute(j + 1, 1)

        return c

    lax.fori_loop(0, (nu + 1) // 2, _pair, 0)


@jax.jit
def kernel(x, expert_indices, expert_weights, w1_stacked, w2_stacked,
           w3_stacked):
    t, h = x.shape
    e, _, inter = w1_stacked.shape
    k = expert_indices.shape[1]

    eit = expert_indices.astype(jnp.int32).T
    ewt = expert_weights.T

    return pl.pallas_call(
        _moe_kernel,
        in_specs=[
            pl.BlockSpec(memory_space=pltpu.MemorySpace.VMEM),
            pl.BlockSpec(memory_space=pltpu.MemorySpace.VMEM),
            pl.BlockSpec(memory_space=pltpu.MemorySpace.SMEM),
            pl.BlockSpec(memory_space=pltpu.MemorySpace.VMEM),
            pl.BlockSpec(memory_space=pltpu.MemorySpace.HBM),
            pl.BlockSpec(memory_space=pltpu.MemorySpace.HBM),
            pl.BlockSpec(memory_space=pltpu.MemorySpace.HBM),
        ],
        out_specs=pl.BlockSpec(memory_space=pltpu.MemorySpace.VMEM),
        out_shape=jax.ShapeDtypeStruct((t, h), jnp.float32),
        scratch_shapes=[
            pltpu.VMEM((e, t), jnp.float32),
            pltpu.SMEM((1, e), jnp.int32),
            pltpu.SMEM((1, e), jnp.int32),
            pltpu.SMEM((1,), jnp.int32),
            pltpu.VMEM((2, h, inter), jnp.float32),
            pltpu.VMEM((2, inter, h), jnp.float32),
            pltpu.VMEM((2, h, inter), jnp.float32),
            pltpu.SemaphoreType.DMA((6,)),
        ],
    )(eit, ewt, eit, x, w1_stacked, w2_stacked, w3_stacked)
